# baseline (speedup 1.0000x reference)
import jax
import jax.numpy as jnp
from jax.experimental import pallas as pl
from jax.experimental.pallas import tpu as pltpu

N = 10000
B = 2
D = 128
H1 = 32
H2 = 32
BM = 200


def _support1_body(state_ref, w1_ref, b1_ref, s1_ref):
    x0 = state_ref[0]
    x1 = state_ref[1]
    s0 = jnp.dot(x0, w1_ref[...], preferred_element_type=jnp.float32)
    s1 = jnp.dot(x1, w1_ref[...], preferred_element_type=jnp.float32)
    s1_ref[...] = jnp.concatenate([s0, s1], axis=1) + b1_ref[...]


def _pass1_body(adj_ref, s1_ref, w2_ref, b2_ref, s2_ref):
    out1 = jnp.dot(adj_ref[...], s1_ref[...], preferred_element_type=jnp.float32)
    out1 = jnp.maximum(out1, 0.0)
    s2_ref[...] = (
        jnp.dot(out1, w2_ref[...], preferred_element_type=jnp.float32) + b2_ref[...]
    )


def _pass2_body(adj_ref, s2_ref, wout_ref, y_ref):
    out2 = jnp.dot(adj_ref[...], s2_ref[...], preferred_element_type=jnp.float32)
    out2 = jnp.maximum(out2, 0.0)
    y_ref[...] = jnp.dot(out2, wout_ref[...], preferred_element_type=jnp.float32)


def kernel(state, adj, W1, b1, W2, b2, Wout):
    f32 = jnp.float32
    b1t = jnp.concatenate([b1, b1]).reshape(1, B * H1).astype(f32)
    b2t = jnp.concatenate([b2, b2]).reshape(1, B * H2).astype(f32)
    z = jnp.zeros((H1, H2), f32)
    w2bd = jnp.block([[W2, z], [z, W2]])
    zo = jnp.zeros((H2, 1), f32)
    woutbd = jnp.block([[Wout, zo], [zo, Wout]])

    s1 = pl.pallas_call(
        _support1_body,
        grid=(1,),
        in_specs=[
            pl.BlockSpec((B, N, D), lambda i: (0, 0, 0)),
            pl.BlockSpec((D, H1), lambda i: (0, 0)),
            pl.BlockSpec((1, B * H1), lambda i: (0, 0)),
        ],
        out_specs=pl.BlockSpec((N, B * H1), lambda i: (0, 0)),
        out_shape=jax.ShapeDtypeStruct((N, B * H1), f32),
    )(state, W1, b1t)

    nblk = N // BM
    row_spec = pl.BlockSpec((BM, N), lambda i: (i, 0))
    full64 = pl.BlockSpec((N, B * H1), lambda i: (0, 0))
    par = pltpu.CompilerParams(
        dimension_semantics=(pltpu.PARALLEL,),
    )

    s2 = pl.pallas_call(
        _pass1_body,
        grid=(nblk,),
        in_specs=[
            row_spec,
            full64,
            pl.BlockSpec((B * H1, B * H2), lambda i: (0, 0)),
            pl.BlockSpec((1, B * H2), lambda i: (0, 0)),
        ],
        out_specs=pl.BlockSpec((BM, B * H2), lambda i: (i, 0)),
        out_shape=jax.ShapeDtypeStruct((N, B * H2), f32),
        compiler_params=par,
    )(adj, s1, w2bd, b2t)

    y = pl.pallas_call(
        _pass2_body,
        grid=(nblk,),
        in_specs=[
            row_spec,
            full64,
            pl.BlockSpec((B * H2, B), lambda i: (0, 0)),
        ],
        out_specs=pl.BlockSpec((BM, B), lambda i: (i, 0)),
        out_shape=jax.ShapeDtypeStruct((N, B), f32),
        compiler_params=par,
    )(adj, s2, woutbd)

    return y.T


# speedup vs baseline: 1.0210x; 1.0210x over previous
import jax
import jax.numpy as jnp
from jax.experimental import pallas as pl
from jax.experimental.pallas import tpu as pltpu

N = 10000
B = 2
D = 128
H1 = 32
H2 = 32
BM = 400


def _support1_body(state_ref, w1_ref, b1_ref, s1_ref):
    x0 = state_ref[0]
    x1 = state_ref[1]
    s0 = jnp.dot(x0, w1_ref[...], preferred_element_type=jnp.float32)
    s1 = jnp.dot(x1, w1_ref[...], preferred_element_type=jnp.float32)
    s1_ref[...] = jnp.concatenate([s0, s1], axis=1) + b1_ref[...]


def _pass1_body(adj_ref, s1_ref, w2_ref, b2_ref, s2_ref):
    out1 = jnp.dot(adj_ref[...], s1_ref[...], preferred_element_type=jnp.float32)
    out1 = jnp.maximum(out1, 0.0)
    s2_ref[...] = (
        jnp.dot(out1, w2_ref[...], preferred_element_type=jnp.float32) + b2_ref[...]
    )


def _pass2_body(adj_ref, s2_ref, wout_ref, y_ref):
    out2 = jnp.dot(adj_ref[...], s2_ref[...], preferred_element_type=jnp.float32)
    out2 = jnp.maximum(out2, 0.0)
    y_ref[...] = jnp.dot(out2, wout_ref[...], preferred_element_type=jnp.float32)


def kernel(state, adj, W1, b1, W2, b2, Wout):
    f32 = jnp.float32
    b1t = jnp.concatenate([b1, b1]).reshape(1, B * H1).astype(f32)
    b2t = jnp.concatenate([b2, b2]).reshape(1, B * H2).astype(f32)
    z = jnp.zeros((H1, H2), f32)
    w2bd = jnp.block([[W2, z], [z, W2]])
    zo = jnp.zeros((H2, 1), f32)
    woutbd = jnp.block([[Wout, zo], [zo, Wout]])

    s1 = pl.pallas_call(
        _support1_body,
        grid=(1,),
        in_specs=[
            pl.BlockSpec((B, N, D), lambda i: (0, 0, 0)),
            pl.BlockSpec((D, H1), lambda i: (0, 0)),
            pl.BlockSpec((1, B * H1), lambda i: (0, 0)),
        ],
        out_specs=pl.BlockSpec((N, B * H1), lambda i: (0, 0)),
        out_shape=jax.ShapeDtypeStruct((N, B * H1), f32),
    )(state, W1, b1t)

    nblk = N // BM
    row_spec = pl.BlockSpec((BM, N), lambda i: (i, 0))
    full64 = pl.BlockSpec((N, B * H1), lambda i: (0, 0))
    par = pltpu.CompilerParams(
        dimension_semantics=(pltpu.PARALLEL,),
    )

    s2 = pl.pallas_call(
        _pass1_body,
        grid=(nblk,),
        in_specs=[
            row_spec,
            full64,
            pl.BlockSpec((B * H1, B * H2), lambda i: (0, 0)),
            pl.BlockSpec((1, B * H2), lambda i: (0, 0)),
        ],
        out_specs=pl.BlockSpec((BM, B * H2), lambda i: (i, 0)),
        out_shape=jax.ShapeDtypeStruct((N, B * H2), f32),
        compiler_params=par,
    )(adj, s1, w2bd, b2t)

    y = pl.pallas_call(
        _pass2_body,
        grid=(nblk,),
        in_specs=[
            row_spec,
            full64,
            pl.BlockSpec((B * H2, B), lambda i: (0, 0)),
        ],
        out_specs=pl.BlockSpec((BM, B), lambda i: (i, 0)),
        out_shape=jax.ShapeDtypeStruct((N, B), f32),
        compiler_params=par,
    )(adj, s2, woutbd)

    return y.T


# speedup vs baseline: 1.0226x; 1.0015x over previous
import jax
import jax.numpy as jnp
from jax.experimental import pallas as pl
from jax.experimental.pallas import tpu as pltpu

N = 10000
B = 2
D = 128
H1 = 32
H2 = 32
BM = 400


def _support1_body(state_ref, w1_ref, b1_ref, s1_ref):
    x0 = state_ref[0]
    x1 = state_ref[1]
    s0 = jnp.dot(x0, w1_ref[...], preferred_element_type=jnp.float32)
    s1 = jnp.dot(x1, w1_ref[...], preferred_element_type=jnp.float32)
    s1_ref[...] = jnp.concatenate([s0, s1], axis=1) + b1_ref[...]


def _pass1_body(adj_ref, s1_ref, w2_ref, b2_ref, s2_ref):
    out1 = jnp.dot(adj_ref[...], s1_ref[...], preferred_element_type=jnp.float32)
    out1 = jnp.maximum(out1, 0.0)
    s2_ref[...] = (
        jnp.dot(out1, w2_ref[...], preferred_element_type=jnp.float32) + b2_ref[...]
    )


def _pass2_body(adj_ref, s2_ref, wout_ref, y_ref):
    out2 = jnp.dot(adj_ref[...], s2_ref[...], preferred_element_type=jnp.float32)
    out2 = jnp.maximum(out2, 0.0)
    y_ref[...] = jnp.dot(out2, wout_ref[...], preferred_element_type=jnp.float32)


def kernel(state, adj, W1, b1, W2, b2, Wout):
    f32 = jnp.float32
    b1t = jnp.concatenate([b1, b1]).reshape(1, B * H1).astype(f32)
    b2t = jnp.concatenate([b2, b2]).reshape(1, B * H2).astype(f32)
    z = jnp.zeros((H1, H2), f32)
    w2bd = jnp.block([[W2, z], [z, W2]])
    zo = jnp.zeros((H2, 1), f32)
    woutbd = jnp.block([[Wout, zo], [zo, Wout]])

    s1 = pl.pallas_call(
        _support1_body,
        grid=(1,),
        in_specs=[
            pl.BlockSpec((B, N, D), lambda i: (0, 0, 0)),
            pl.BlockSpec((D, H1), lambda i: (0, 0)),
            pl.BlockSpec((1, B * H1), lambda i: (0, 0)),
        ],
        out_specs=pl.BlockSpec((N, B * H1), lambda i: (0, 0)),
        out_shape=jax.ShapeDtypeStruct((N, B * H1), f32),
    )(state, W1, b1t)

    nblk = N // BM
    row_spec = pl.BlockSpec((BM, N), lambda i: (i, 0))
    full64 = pl.BlockSpec((N, B * H1), lambda i: (0, 0))
    par = pltpu.CompilerParams(
        dimension_semantics=(pltpu.ARBITRARY,),
    )

    s2 = pl.pallas_call(
        _pass1_body,
        grid=(nblk,),
        in_specs=[
            row_spec,
            full64,
            pl.BlockSpec((B * H1, B * H2), lambda i: (0, 0)),
            pl.BlockSpec((1, B * H2), lambda i: (0, 0)),
        ],
        out_specs=pl.BlockSpec((BM, B * H2), lambda i: (i, 0)),
        out_shape=jax.ShapeDtypeStruct((N, B * H2), f32),
        compiler_params=par,
    )(adj, s1, w2bd, b2t)

    y = pl.pallas_call(
        _pass2_body,
        grid=(nblk,),
        in_specs=[
            row_spec,
            full64,
            pl.BlockSpec((B * H2, B), lambda i: (0, 0)),
        ],
        out_specs=pl.BlockSpec((BM, B), lambda i: (i, 0)),
        out_shape=jax.ShapeDtypeStruct((N, B), f32),
        compiler_params=par,
    )(adj, s2, woutbd)

    return y.T


# speedup vs baseline: 1.0695x; 1.0459x over previous
import jax
import jax.numpy as jnp
from jax.experimental import pallas as pl
from jax.experimental.pallas import tpu as pltpu

N = 10000
B = 2
D = 128
H1 = 32
H2 = 32
BM = 400


def _fused_body(state_ref, adj_ref, w1_ref, b1_ref, w2_ref, b2_ref, wout_ref,
                y_ref, sa_ref, sb_ref):
    p = pl.program_id(0)
    i = pl.program_id(1)

    @pl.when((p == 0) & (i == 0))
    def _init():
        x0 = state_ref[0]
        x1 = state_ref[1]
        s0 = jnp.dot(x0, w1_ref[...], preferred_element_type=jnp.float32)
        s1 = jnp.dot(x1, w1_ref[...], preferred_element_type=jnp.float32)
        sa_ref[...] = jnp.concatenate([s0, s1], axis=1) + b1_ref[...]

    @pl.when(p == 0)
    def _pass0():
        out1 = jnp.dot(adj_ref[...], sa_ref[...],
                       preferred_element_type=jnp.float32)
        out1 = jnp.maximum(out1, 0.0)
        sb_ref[pl.ds(i * BM, BM), :] = (
            jnp.dot(out1, w2_ref[...], preferred_element_type=jnp.float32)
            + b2_ref[...]
        )

    @pl.when(p == 1)
    def _pass1():
        out2 = jnp.dot(adj_ref[...], sb_ref[...],
                       preferred_element_type=jnp.float32)
        out2 = jnp.maximum(out2, 0.0)
        y_ref[...] = jnp.dot(out2, wout_ref[...],
                             preferred_element_type=jnp.float32)


def kernel(state, adj, W1, b1, W2, b2, Wout):
    f32 = jnp.float32
    b1t = jnp.concatenate([b1, b1]).reshape(1, B * H1).astype(f32)
    b2t = jnp.concatenate([b2, b2]).reshape(1, B * H2).astype(f32)
    z = jnp.zeros((H1, H2), f32)
    w2bd = jnp.block([[W2, z], [z, W2]])
    zo = jnp.zeros((H2, 1), f32)
    woutbd = jnp.block([[Wout, zo], [zo, Wout]])

    nblk = N // BM
    y = pl.pallas_call(
        _fused_body,
        grid=(2, nblk),
        in_specs=[
            pl.BlockSpec((B, N, D), lambda p, i: (0, 0, 0)),
            pl.BlockSpec((BM, N), lambda p, i: (i, 0)),
            pl.BlockSpec((D, H1), lambda p, i: (0, 0)),
            pl.BlockSpec((1, B * H1), lambda p, i: (0, 0)),
            pl.BlockSpec((B * H1, B * H2), lambda p, i: (0, 0)),
            pl.BlockSpec((1, B * H2), lambda p, i: (0, 0)),
            pl.BlockSpec((B * H2, B), lambda p, i: (0, 0)),
        ],
        out_specs=pl.BlockSpec((BM, B), lambda p, i: (i, 0)),
        out_shape=jax.ShapeDtypeStruct((N, B), f32),
        scratch_shapes=[
            pltpu.VMEM((N, B * H1), f32),
            pltpu.VMEM((N, B * H2), f32),
        ],
    )(state, adj, W1, b1t, w2bd, b2t, woutbd)

    return y.T
